# baseline (speedup 1.0000x reference)
import functools

import jax
import jax.numpy as jnp
from jax.experimental import pallas as pl

N = 10000
E = 320000
D_IN = 128
H = 256
OUT = 64
EPS = 1e-07
TOTAL_COUNT = 10000.0


def _mm_body(h_ref, w_ref, o_ref):
    o_ref[...] = jnp.dot(h_ref[...], w_ref[...],
                         preferred_element_type=jnp.float32)


def _matmul(h, W, blk=2000):
    n, d = h.shape
    do = W.shape[1]
    grid = (n // blk,)
    return pl.pallas_call(
        _mm_body,
        grid=grid,
        in_specs=[
            pl.BlockSpec((blk, d), lambda i: (i, 0)),
            pl.BlockSpec((d, do), lambda i: (0, 0)),
        ],
        out_specs=pl.BlockSpec((blk, do), lambda i: (i, 0)),
        out_shape=jax.ShapeDtypeStruct((n, do), jnp.float32),
    )(h, W)


def kernel(x, edge_index, W0, att_src0, att_dst0, gamma0, beta0,
           W1, att_src1, att_dst1, gamma1, beta1, W_loc, b_loc, W_std, b_std):
    l = jnp.sum(x, axis=1, keepdims=True)
    h = jnp.log1p(x * (TOTAL_COUNT / l))
    src, dst = edge_index[0], edge_index[1]
    for (W, a_s, a_d, g, b) in [(W0, att_src0, att_dst0, gamma0, beta0),
                                (W1, att_src1, att_dst1, gamma1, beta1)]:
        hp = _matmul(h, W)
        alpha_src = hp @ a_s
        alpha_dst = hp @ a_d
        e = jax.nn.leaky_relu(alpha_src[src] + alpha_dst[dst],
                              negative_slope=0.2)
        m = jax.ops.segment_max(e, dst, num_segments=N)
        m = jnp.where(jnp.isfinite(m), m, 0.0)
        ex = jnp.exp(e - m[dst])
        den = jax.ops.segment_sum(ex, dst, num_segments=N)
        alpha = ex / (den[dst] + 1e-16)
        h = jax.ops.segment_sum(alpha[:, None] * hp[src], dst, num_segments=N)
        mu = jnp.mean(h, axis=0)
        var = jnp.mean((h - mu) ** 2, axis=0)
        h = g * (h - mu) / jnp.sqrt(var + 1e-05) + b
        h = jax.nn.leaky_relu(h, negative_slope=0.2)
    loc = h @ W_loc + b_loc
    std = jax.nn.softplus(h @ W_std + b_std) + EPS
    return (loc, std, l)


# speedup vs baseline: 13.3753x; 13.3753x over previous
import functools

import jax
import jax.numpy as jnp
from jax import lax
from jax.experimental import pallas as pl
from jax.experimental.pallas import tpu as pltpu
from jax.experimental.pallas import tpu_sc as plsc

N = 10000
NP = 10240
E = 320000
D_IN = 128
H = 256
OUT = 64
EPS = 1e-07
TOTAL_COUNT = 10000.0

NS = 16
EPT = E // NS
GB = 128
NCH = 160
EPT_P = NCH * GB
DEN_R = NP // 128
BLK = 2000
GRID = N // BLK

_f32 = jnp.float32
_i32 = jnp.int32
_bf16 = jnp.bfloat16


def _dot(a, b):
    return jnp.dot(a.astype(_bf16), b.astype(_bf16),
                   preferred_element_type=_f32)


def _k1_body(x_ref, w_ref, a_ref, l_ref, hp_ref, as_ref, ad_ref, mm_ref):
    i = pl.program_id(0)
    xb = x_ref[...]
    lb = jnp.sum(xb, axis=1, keepdims=True)
    hb = jnp.log1p(xb * (TOTAL_COUNT / lb))
    l_ref[...] = lb
    hp0 = _dot(hb, w_ref[0])
    hp1 = _dot(hb, w_ref[1])
    hp_ref[0] = hp0
    hp_ref[1] = hp1
    asb = _dot(hp0, a_ref[0, 0]) + _dot(hp1, a_ref[0, 1])
    adb = _dot(hp0, a_ref[1, 0]) + _dot(hp1, a_ref[1, 1])
    as_ref[...] = asb[:, None]
    ad_ref[...] = adb[:, None]
    cur = jnp.concatenate([jnp.max(asb).reshape(1, 1),
                           jnp.max(adb).reshape(1, 1)], axis=1)

    @pl.when(i == 0)
    def _():
        mm_ref[...] = cur

    @pl.when(i > 0)
    def _():
        mm_ref[...] = jnp.maximum(mm_ref[...], cur)


def _k1(x, w_split, a_split):
    return pl.pallas_call(
        _k1_body,
        grid=(GRID,),
        in_specs=[
            pl.BlockSpec((BLK, D_IN), lambda i: (i, 0)),
            pl.BlockSpec((2, D_IN, 128), lambda i: (0, 0, 0)),
            pl.BlockSpec((2, 2, 128), lambda i: (0, 0, 0)),
        ],
        out_specs=[
            pl.BlockSpec((BLK, 1), lambda i: (i, 0)),
            pl.BlockSpec((2, BLK, 128), lambda i: (0, i, 0)),
            pl.BlockSpec((BLK, 1), lambda i: (i, 0)),
            pl.BlockSpec((BLK, 1), lambda i: (i, 0)),
            pl.BlockSpec((1, 2), lambda i: (0, 0)),
        ],
        out_shape=[
            jax.ShapeDtypeStruct((N, 1), _f32),
            jax.ShapeDtypeStruct((2, N, 128), _f32),
            jax.ShapeDtypeStruct((N, 1), _f32),
            jax.ShapeDtypeStruct((N, 1), _f32),
            jax.ShapeDtypeStruct((1, 2), _f32),
        ],
    )(x, w_split, a_split)


def _stats_body(a_ref, s_ref, q_ref):
    i = pl.program_id(0)
    ab = a_ref[...]
    s = jnp.sum(ab, axis=1)
    q = jnp.sum(ab * ab, axis=1)

    @pl.when(i == 0)
    def _():
        s_ref[...] = s
        q_ref[...] = q

    @pl.when(i > 0)
    def _():
        s_ref[...] += s
        q_ref[...] += q


def _stats(agg):
    return pl.pallas_call(
        _stats_body,
        grid=(GRID,),
        in_specs=[pl.BlockSpec((2, BLK, 128), lambda i: (0, i, 0))],
        out_specs=[
            pl.BlockSpec((2, 128), lambda i: (0, 0)),
            pl.BlockSpec((2, 128), lambda i: (0, 0)),
        ],
        out_shape=[
            jax.ShapeDtypeStruct((2, 128), _f32),
            jax.ShapeDtypeStruct((2, 128), _f32),
        ],
    )(agg)


def _bn_block(ab, s_ref, q_ref, g_ref, b_ref):
    mu = s_ref[...] * (1.0 / N)
    var = q_ref[...] * (1.0 / N) - mu * mu
    scale = g_ref[...] * lax.rsqrt(var + 1e-05)
    hn = (ab - mu[:, None, :]) * scale[:, None, :] + b_ref[...][:, None, :]
    hn = jnp.where(hn >= 0.0, hn, 0.2 * hn)
    return jnp.concatenate([hn[0], hn[1]], axis=1)


def _mm1_body(a_in_ref, s_ref, q_ref, g_ref, b_ref, w_ref, av_ref,
              hp_ref, as_ref, ad_ref, mm_ref):
    i = pl.program_id(0)
    hfull = _bn_block(a_in_ref[...], s_ref, q_ref, g_ref, b_ref)
    hp0 = _dot(hfull, w_ref[0])
    hp1 = _dot(hfull, w_ref[1])
    hp_ref[0] = hp0
    hp_ref[1] = hp1
    asb = _dot(hp0, av_ref[0, 0]) + _dot(hp1, av_ref[0, 1])
    adb = _dot(hp0, av_ref[1, 0]) + _dot(hp1, av_ref[1, 1])
    as_ref[...] = asb[:, None]
    ad_ref[...] = adb[:, None]
    cur = jnp.concatenate([jnp.max(asb).reshape(1, 1),
                           jnp.max(adb).reshape(1, 1)], axis=1)

    @pl.when(i == 0)
    def _():
        mm_ref[...] = cur

    @pl.when(i > 0)
    def _():
        mm_ref[...] = jnp.maximum(mm_ref[...], cur)


def _mm1(agg, s, q, g2, b2, w_split, a_split):
    return pl.pallas_call(
        _mm1_body,
        grid=(GRID,),
        in_specs=[
            pl.BlockSpec((2, BLK, 128), lambda i: (0, i, 0)),
            pl.BlockSpec((2, 128), lambda i: (0, 0)),
            pl.BlockSpec((2, 128), lambda i: (0, 0)),
            pl.BlockSpec((2, 128), lambda i: (0, 0)),
            pl.BlockSpec((2, 128), lambda i: (0, 0)),
            pl.BlockSpec((2, H, 128), lambda i: (0, 0, 0)),
            pl.BlockSpec((2, 2, 128), lambda i: (0, 0, 0)),
        ],
        out_specs=[
            pl.BlockSpec((2, BLK, 128), lambda i: (0, i, 0)),
            pl.BlockSpec((BLK, 1), lambda i: (i, 0)),
            pl.BlockSpec((BLK, 1), lambda i: (i, 0)),
            pl.BlockSpec((1, 2), lambda i: (0, 0)),
        ],
        out_shape=[
            jax.ShapeDtypeStruct((2, N, 128), _f32),
            jax.ShapeDtypeStruct((N, 1), _f32),
            jax.ShapeDtypeStruct((N, 1), _f32),
            jax.ShapeDtypeStruct((1, 2), _f32),
        ],
    )(agg, s, q, g2, b2, w_split, a_split)


def _heads_body(a_in_ref, s_ref, q_ref, g_ref, b_ref,
                wl_ref, bl_ref, ws_ref, bs_ref, loc_ref, std_ref):
    hfull = _bn_block(a_in_ref[...], s_ref, q_ref, g_ref, b_ref)
    loc_ref[...] = _dot(hfull, wl_ref[...]) + bl_ref[...]
    zs = _dot(hfull, ws_ref[...]) + bs_ref[...]
    std_ref[...] = jax.nn.softplus(zs) + EPS


def _heads(agg, s, q, g2, b2, wl, bl, ws, bs):
    return pl.pallas_call(
        _heads_body,
        grid=(GRID,),
        in_specs=[
            pl.BlockSpec((2, BLK, 128), lambda i: (0, i, 0)),
            pl.BlockSpec((2, 128), lambda i: (0, 0)),
            pl.BlockSpec((2, 128), lambda i: (0, 0)),
            pl.BlockSpec((2, 128), lambda i: (0, 0)),
            pl.BlockSpec((2, 128), lambda i: (0, 0)),
            pl.BlockSpec((H, OUT), lambda i: (0, 0)),
            pl.BlockSpec((1, OUT), lambda i: (0, 0)),
            pl.BlockSpec((H, OUT), lambda i: (0, 0)),
            pl.BlockSpec((1, OUT), lambda i: (0, 0)),
        ],
        out_specs=[
            pl.BlockSpec((BLK, OUT), lambda i: (i, 0)),
            pl.BlockSpec((BLK, OUT), lambda i: (i, 0)),
        ],
        out_shape=[
            jax.ShapeDtypeStruct((N, OUT), _f32),
            jax.ShapeDtypeStruct((N, OUT), _f32),
        ],
    )(agg, s, q, g2, b2, wl, bl, ws, bs)


def _sc_body(src_hbm, dst_hbm, asrc_hbm, adst_hbm, mub_hbm, hp_hbm,
             out_hbm, ex_hbm, den_sh, acc,
             sia, sib, sea, seb, sga, sgb, sda, sdb):
    c = lax.axis_index("c")
    t = lax.axis_index("s")
    cn = c * N

    def main(denv, ra, rb, srcb, dstb, exb, ia, ib, ea, eb, da, db,
             idv, mubv):
        _sc_tile(src_hbm, dst_hbm, asrc_hbm, adst_hbm, mub_hbm, hp_hbm,
                 out_hbm, ex_hbm, den_sh, acc,
                 (sia, sib, sea, seb, sga, sgb, sda, sdb),
                 denv, ra, rb, srcb, dstb, exb, ia, ib, ea, eb, da, db,
                 idv, mubv, c, t, cn)

    pl.run_scoped(
        main,
        pltpu.VMEM((DEN_R, 128), _f32),
        pltpu.VMEM((GB, 128), _f32),
        pltpu.VMEM((GB, 128), _f32),
        pltpu.VMEM((10, GB), _i32),
        pltpu.VMEM((10, GB), _i32),
        pltpu.VMEM((10, GB), _f32),
        pltpu.VMEM((GB,), _i32),
        pltpu.VMEM((GB,), _i32),
        pltpu.VMEM((GB,), _f32),
        pltpu.VMEM((GB,), _f32),
        pltpu.VMEM((GB,), _i32),
        pltpu.VMEM((GB,), _i32),
        pltpu.VMEM((DEN_R,), _i32),
        pltpu.VMEM((16,), _f32),
    )


def _sc_tile(src_hbm, dst_hbm, asrc_hbm, adst_hbm, mub_hbm, hp_hbm,
             out_hbm, ex_hbm, den_sh, acc, sems,
             denv, ra, rb, srcb, dstb, exb, ia, ib, ea, eb, da, db,
             idv, mubv, c, t, cn):
    sia, sib, sea, seb, sga, sgb, sda, sdb = sems
    zero16 = jnp.zeros((16,), _f32)

    pltpu.sync_copy(asrc_hbm, ra.at[pl.ds(0, DEN_R)])
    pltpu.sync_copy(adst_hbm, rb.at[pl.ds(0, DEN_R)])
    pltpu.sync_copy(mub_hbm, mubv)
    mub = mubv[...]

    @pl.loop(0, DEN_R)
    def _(r):
        for i in range(8):
            denv[r, pl.ds(i * 16, 16)] = zero16

    @pl.loop(0, DEN_R // 16)
    def _(i):
        idv[pl.ds(i * 16, 16)] = lax.iota(_i32, 16) + i * 16

    @pl.when(t == 0)
    def _():
        pltpu.sync_copy(denv, den_sh)

    @pl.loop(0, NCH // 10)
    def _(sg):
        pltpu.sync_copy(src_hbm.at[t, pl.ds(sg * 10, 10)], srcb)
        pltpu.sync_copy(dst_hbm.at[t, pl.ds(sg * 10, 10)], dstb)

        @pl.loop(0, 10)
        def _(r):
            for k in range(GB // 16):
                sl = pl.ds(k * 16, 16)
                s16 = srcb[r, sl]
                d16 = dstb[r, sl]
                dr = jnp.right_shift(d16, 7)
                dc = jnp.bitwise_and(d16, 127)
                z = (plsc.load_gather(ra, [jnp.right_shift(s16, 7),
                                           jnp.bitwise_and(s16, 127)])
                     + plsc.load_gather(rb, [dr, dc]))
                e = jnp.where(z >= 0.0, z, 0.2 * z)
                ex = jnp.exp(jnp.maximum(e - mub, -80.0))
                exb[r, sl] = ex
                plsc.addupdate_scatter(denv, [dr, dc], ex)

        pltpu.sync_copy(exb, ex_hbm.at[c, t, pl.ds(sg * 10, 10)])

    plsc.subcore_barrier()
    pltpu.sync_copy(denv, den_sh.at[idv], add=True)
    plsc.subcore_barrier()
    pltpu.sync_copy(den_sh, denv)

    @pl.loop(0, GB)
    def _(r):
        for i in range(8):
            ra[r, pl.ds(i * 16, 16)] = zero16

    r0 = t * (NP // NS)

    @pl.loop(0, NP // NS // GB)
    def _(i):
        pltpu.sync_copy(ra, acc.at[pl.ds(r0 + i * GB, GB)])

    plsc.subcore_barrier()

    def stage_idx(g, ibuf, sem):
        gc = jnp.minimum(g, NCH - 1)
        pltpu.async_copy(src_hbm.at[t, gc], ibuf, sem)

    def wait_idx(ibuf, sem):
        pltpu.make_async_copy(src_hbm.at[0, 0], ibuf, sem).wait()
        for i in range(GB // 16):
            sl = pl.ds(i * 16, 16)
            ibuf[sl] = ibuf[sl] + cn

    def stage_ex(g, ebuf, sem):
        gc = jnp.minimum(g, NCH - 1)
        pltpu.async_copy(ex_hbm.at[c, t, gc], ebuf, sem)

    def wait_ex(ebuf, sem):
        pltpu.make_async_copy(ex_hbm.at[0, 0, 0], ebuf, sem).wait()

    def stage_dst(g, dbuf, sem):
        gc = jnp.minimum(g, NCH - 1)
        pltpu.async_copy(dst_hbm.at[t, gc], dbuf, sem)

    def wait_dst(dbuf, sem):
        pltpu.make_async_copy(dst_hbm.at[0, 0], dbuf, sem).wait()

    def gather(ibuf, buf, sem):
        pltpu.async_copy(hp_hbm.at[ibuf], buf, sem)

    def wait_rows(buf, sem):
        pltpu.make_async_copy(hp_hbm.at[pl.ds(0, GB)], buf, sem).wait()

    def process(buf, ebuf, dbuf):
        for k in range(GB // 16):
            sl = pl.ds(k * 16, 16)
            d16 = dbuf[sl]
            dn = plsc.load_gather(denv, [jnp.right_shift(d16, 7),
                                         jnp.bitwise_and(d16, 127)])
            ebuf[sl] = ebuf[sl] / dn

        @pl.loop(0, GB, unroll=8)
        def _(r):
            av = plsc.load_gather(ebuf, [jnp.zeros((16,), _i32) + r])
            for i in range(8):
                sl = pl.ds(i * 16, 16)
                buf[r, sl] = buf[r, sl] * av

        pltpu.sync_copy(buf, acc.at[dbuf], add=True)

    stage_idx(0, ia, sia)
    stage_ex(0, ea, sea)
    stage_dst(0, da, sda)
    wait_idx(ia, sia)
    gather(ia, ra, sga)
    stage_idx(1, ib, sib)
    stage_ex(1, eb, seb)
    stage_dst(1, db, sdb)

    @pl.loop(0, NCH // 2)
    def _(gg):
        g0 = gg * 2
        wait_rows(ra, sga)
        wait_idx(ib, sib)
        gather(ib, rb, sgb)
        stage_idx(g0 + 2, ia, sia)
        wait_ex(ea, sea)
        wait_dst(da, sda)
        process(ra, ea, da)
        stage_ex(g0 + 2, ea, sea)
        stage_dst(g0 + 2, da, sda)
        wait_rows(rb, sgb)
        wait_idx(ia, sia)
        gather(ia, ra, sga)
        stage_idx(g0 + 3, ib, sib)
        wait_ex(eb, seb)
        wait_dst(db, sdb)
        process(rb, eb, db)
        stage_ex(g0 + 3, eb, seb)
        stage_dst(g0 + 3, db, sdb)

    wait_rows(ra, sga)
    wait_idx(ib, sib)
    wait_ex(ea, sea)
    wait_ex(eb, seb)
    wait_dst(da, sda)
    wait_dst(db, sdb)

    plsc.subcore_barrier()

    @pl.loop(0, NP // NS // GB)
    def _(i):
        pltpu.sync_copy(acc.at[pl.ds(r0 + i * GB, GB)], ra)
        pltpu.sync_copy(ra, out_hbm.at[pl.ds(c * NP + r0 + i * GB, GB)])


def _sc_layer(srcp, dst3, asrc_p, adst_p, mub, hp_cat):
    mesh = plsc.VectorSubcoreMesh(core_axis_name="c", subcore_axis_name="s")
    out, _ = pl.kernel(
        _sc_body,
        out_type=[
            jax.ShapeDtypeStruct((2 * NP, 128), _f32),
            jax.ShapeDtypeStruct((2, NS, NCH, GB), _f32),
        ],
        mesh=mesh,
        compiler_params=pltpu.CompilerParams(needs_layout_passes=False,
                                             use_tc_tiling_on_sc=False),
        scratch_types=[
            pltpu.VMEM_SHARED((DEN_R, 128), _f32),
            pltpu.VMEM_SHARED((NP, 128), _f32),
            pltpu.SemaphoreType.DMA,
            pltpu.SemaphoreType.DMA,
            pltpu.SemaphoreType.DMA,
            pltpu.SemaphoreType.DMA,
            pltpu.SemaphoreType.DMA,
            pltpu.SemaphoreType.DMA,
            pltpu.SemaphoreType.DMA,
            pltpu.SemaphoreType.DMA,
        ],
    )(srcp, dst3, asrc_p, adst_p, mub, hp_cat)
    return out


def _split_w(W):
    di = W.shape[0]
    return jnp.transpose(W.reshape(di, 2, 128), (1, 0, 2))


def _split_feat(v):
    return v.reshape(2, 128)


def _split_att(a_s, a_d):
    return jnp.stack([a_s.reshape(2, 128), a_d.reshape(2, 128)])


def kernel(x, edge_index, W0, att_src0, att_dst0, gamma0, beta0,
           W1, att_src1, att_dst1, gamma1, beta1, W_loc, b_loc, W_std, b_std):
    pad = EPT_P - EPT
    srcp = jnp.pad(edge_index[0].reshape(NS, EPT),
                   ((0, 0), (0, pad))).reshape(NS, NCH, GB)
    dst3 = jnp.pad(edge_index[1].reshape(NS, EPT), ((0, 0), (0, pad)),
                   constant_values=N).reshape(NS, NCH, GB)

    def _padN(v):
        return jnp.pad(v.reshape(N), (0, DEN_R * 128 - N)).reshape(DEN_R, 128)

    l, hp_split, asrc, adst, mm = _k1(x, _split_w(W0),
                                      _split_att(att_src0, att_dst0))
    mub = jnp.broadcast_to(mm[0, 0] + mm[0, 1], (16,))
    agg = _sc_layer(srcp, dst3, _padN(asrc), _padN(adst), mub,
                    hp_split.reshape(2 * N, 128))
    agg = agg.reshape(2, NP, 128)[:, :N, :]

    s0, q0 = _stats(agg)
    hp_split1, asrc1, adst1, mm1 = _mm1(
        agg, s0, q0, _split_feat(gamma0), _split_feat(beta0),
        _split_w(W1), _split_att(att_src1, att_dst1))
    mub1 = jnp.broadcast_to(mm1[0, 0] + mm1[0, 1], (16,))
    agg1 = _sc_layer(srcp, dst3, _padN(asrc1), _padN(adst1), mub1,
                     hp_split1.reshape(2 * N, 128))
    agg1 = agg1.reshape(2, NP, 128)[:, :N, :]

    s1, q1 = _stats(agg1)
    loc, std = _heads(agg1, s1, q1, _split_feat(gamma1), _split_feat(beta1),
                      W_loc, b_loc.reshape(1, OUT), W_std, b_std.reshape(1, OUT))
    return (loc, std, l)
